# baseline (speedup 1.0000x reference)
import functools

import jax
import jax.numpy as jnp
from jax import lax
from jax.experimental import pallas as pl
from jax.experimental.pallas import tpu as pltpu
from jax.experimental.pallas import tpu_sc as plsc

B = 32768
E = 64
EP = E + 1
K = 8
NC, NS = 2, 16
NW = NC * NS
ROWS_PER_W = B // NW
CHUNK = 256
N_CHUNKS = ROWS_PER_W // CHUNK
GROUPS = CHUNK // 16
CAPACITY = int(B * 1.25 / E)
R_BLK = 1024

_NET19 = [(0, 1), (2, 3), (4, 5), (6, 7),
          (0, 2), (1, 3), (4, 6), (5, 7),
          (1, 2), (5, 6),
          (0, 4), (1, 5), (2, 6), (3, 7),
          (2, 4), (3, 5),
          (1, 2), (3, 4), (5, 6)]
_BITONIC12 = [(0, 4), (1, 5), (2, 6), (3, 7),
              (0, 2), (1, 3), (4, 6), (5, 7),
              (0, 1), (2, 3), (4, 5), (6, 7)]


def _router_body(gates_hbm, rw_hbm, tki_hbm, tkw_hbm, pcnt_hbm,
                 gbuf0, gbuf1, gbufp, rwbuf0, rwbuf1, tki0, tki1,
                 tkw0, tkw1, cntbuf,
                 isem0, isem1, rsem0, rsem1, ksem0, ksem1, wsem0, wsem1):
    wid = lax.axis_index("s") * NC + lax.axis_index("c")
    lane = lax.iota(jnp.int32, 16)
    zf = jnp.zeros((16,), jnp.float32)
    zi = jnp.zeros((16,), jnp.int32)
    gbufs, rwbufs = [gbuf0, gbuf1], [rwbuf0, rwbuf1]
    tkibufs, tkwbufs = [tki0, tki1], [tkw0, tkw1]
    isems, rsems = [isem0, isem1], [rsem0, rsem1]
    ksems, wsems = [ksem0, ksem1], [wsem0, wsem1]

    def gslice(c):
        return gates_hbm.at[pl.ds(wid * (ROWS_PER_W * E) + c * (CHUNK * E),
                                  CHUNK * E)]

    def rwslice(c):
        return rw_hbm.at[pl.ds(wid * (ROWS_PER_W * E) + c * (CHUNK * E),
                               CHUNK * E)]

    def kslice(ref, c):
        return ref.at[pl.ds(wid * (ROWS_PER_W * K) + c * (CHUNK * K),
                            CHUNK * K)]

    for q in range(E // 16):
        cntbuf[pl.ds(q * 16, 16)] = zi

    pltpu.make_async_copy(gslice(0), gbufs[0], isems[0]).start()

    def process_chunk(c, b, gbuf, rwbuf, tkibuf, tkwbuf):
        def repitch_body(r8, _):
            b64 = r8 * (8 * E)
            b65 = r8 * (8 * EP)
            for i in range(8):
                for q in range(E // 16):
                    gbufp[pl.ds(b65 + i * EP + q * 16, 16)] = (
                        gbuf[pl.ds(b64 + i * E + q * 16, 16)])
            return 0

        lax.fori_loop(0, CHUNK // 8, repitch_body, 0)

        def do_group(g):
            goff = g * (16 * E)
            goffp = g * (16 * EP)
            for r in range(16):
                for q in range(E // 16):
                    rwbuf[pl.ds(goff + r * E + q * 16, 16)] = zf

            rowaddr = goff + lane * E
            rowaddrp = goffp + lane * EP

            def keys_of(t):
                ks = []
                for j in range(8):
                    e = t * 8 + j
                    v = plsc.load_gather(gbufp, [rowaddrp + e])
                    bits = lax.bitcast_convert_type(v, jnp.int32)
                    kb = (bits & -64) | (63 - e)
                    ks.append(lax.bitcast_convert_type(kb, jnp.float32))
                return ks

            def sort8(v):
                for i, j in _NET19:
                    v[i], v[j] = jnp.maximum(v[i], v[j]), jnp.minimum(v[i], v[j])
                return v

            def merge_top8(a, b):
                c8 = [jnp.maximum(a[i], b[7 - i]) for i in range(8)]
                for i, j in _BITONIC12:
                    c8[i], c8[j] = (jnp.maximum(c8[i], c8[j]),
                                    jnp.minimum(c8[i], c8[j]))
                return c8

            m01 = merge_top8(sort8(keys_of(0)), sort8(keys_of(1)))
            m23 = merge_top8(sort8(keys_of(2)), sort8(keys_of(3)))
            m03 = merge_top8(m01, m23)
            m45 = merge_top8(sort8(keys_of(4)), sort8(keys_of(5)))
            m67 = merge_top8(sort8(keys_of(6)), sort8(keys_of(7)))
            m47 = merge_top8(m45, m67)
            f = merge_top8(m03, m47)

            mi = [63 - (lax.bitcast_convert_type(f[j], jnp.int32) & 63)
                  for j in range(K)]
            m = [plsc.load_gather(gbufp, [rowaddrp + mi[j]]) for j in range(K)]

            p = [jnp.exp(m[j] - m[0]) for j in range(K)]
            s = p[0]
            for j in range(1, K):
                s = s + p[j]
            inv = 1.0 / s
            w = [p[j] * inv for j in range(K)]

            trow = g * (16 * K) + lane * K
            one = jnp.ones((16,), jnp.int32)
            for j in range(K):
                plsc.store_scatter(rwbuf, [rowaddr + mi[j]], w[j])
                plsc.store_scatter(tkwbuf, [trow + j], w[j])
                plsc.store_scatter(tkibuf, [trow + j], mi[j])
                plsc.addupdate_scatter(
                    cntbuf, [mi[j]], jnp.where(w[j] > 0.0, one, zi))

        def group_pair_body(g2, _):
            do_group(g2 * 2)
            do_group(g2 * 2 + 1)
            return 0

        lax.fori_loop(0, GROUPS // 2, group_pair_body, 0)

    def outer_body(c2, _):
        for b in range(2):
            c = c2 * 2 + b

            @pl.when(c + 1 < N_CHUNKS)
            def _():
                pltpu.make_async_copy(gslice(c + 1), gbufs[1 - b],
                                      isems[1 - b]).start()

            pltpu.make_async_copy(gslice(c), gbufs[b], isems[b]).wait()

            @pl.when(c >= 2)
            def _():
                pltpu.make_async_copy(rwbufs[b], rwslice(c - 2),
                                      rsems[b]).wait()
                pltpu.make_async_copy(tkibufs[b], kslice(tki_hbm, c - 2),
                                      ksems[b]).wait()
                pltpu.make_async_copy(tkwbufs[b], kslice(tkw_hbm, c - 2),
                                      wsems[b]).wait()

            process_chunk(c, b, gbufs[b], rwbufs[b], tkibufs[b], tkwbufs[b])

            pltpu.make_async_copy(rwbufs[b], rwslice(c), rsems[b]).start()
            pltpu.make_async_copy(tkibufs[b], kslice(tki_hbm, c),
                                  ksems[b]).start()
            pltpu.make_async_copy(tkwbufs[b], kslice(tkw_hbm, c),
                                  wsems[b]).start()
        return 0

    lax.fori_loop(0, N_CHUNKS // 2, outer_body, 0)

    for b in range(2):
        c = N_CHUNKS - 2 + b
        pltpu.make_async_copy(rwbufs[b], rwslice(c), rsems[b]).wait()
        pltpu.make_async_copy(tkibufs[b], kslice(tki_hbm, c), ksems[b]).wait()
        pltpu.make_async_copy(tkwbufs[b], kslice(tkw_hbm, c), wsems[b]).wait()
    pltpu.sync_copy(cntbuf, pcnt_hbm.at[pl.ds(wid * E, E)])


@functools.partial(
    pl.kernel,
    out_type=(
        jax.ShapeDtypeStruct((B * E,), jnp.float32),
        jax.ShapeDtypeStruct((B * K,), jnp.int32),
        jax.ShapeDtypeStruct((B * K,), jnp.float32),
        jax.ShapeDtypeStruct((NW * E,), jnp.int32),
    ),
    mesh=plsc.VectorSubcoreMesh(core_axis_name="c", subcore_axis_name="s"),
    compiler_params=pltpu.CompilerParams(needs_layout_passes=False),
    scratch_types=[
        pltpu.VMEM((CHUNK * E,), jnp.float32),
        pltpu.VMEM((CHUNK * E,), jnp.float32),
        pltpu.VMEM((CHUNK * EP,), jnp.float32),
        pltpu.VMEM((CHUNK * E,), jnp.float32),
        pltpu.VMEM((CHUNK * E,), jnp.float32),
        pltpu.VMEM((CHUNK * K,), jnp.int32),
        pltpu.VMEM((CHUNK * K,), jnp.int32),
        pltpu.VMEM((CHUNK * K,), jnp.float32),
        pltpu.VMEM((CHUNK * K,), jnp.float32),
        pltpu.VMEM((E,), jnp.int32),
        pltpu.SemaphoreType.DMA,
        pltpu.SemaphoreType.DMA,
        pltpu.SemaphoreType.DMA,
        pltpu.SemaphoreType.DMA,
        pltpu.SemaphoreType.DMA,
        pltpu.SemaphoreType.DMA,
        pltpu.SemaphoreType.DMA,
        pltpu.SemaphoreType.DMA,
    ],
)
def _router(gates_hbm, rw_hbm, tki_hbm, tkw_hbm, pcnt_hbm,
            gbuf0, gbuf1, gbufp, rwbuf0, rwbuf1, tki0, tki1,
            tkw0, tkw1, cntbuf,
            isem0, isem1, rsem0, rsem1, ksem0, ksem1, wsem0, wsem1):
    _router_body(gates_hbm, rw_hbm, tki_hbm, tkw_hbm, pcnt_hbm,
                 gbuf0, gbuf1, gbufp, rwbuf0, rwbuf1, tki0, tki1,
                 tkw0, tkw1, cntbuf,
                 isem0, isem1, rsem0, rsem1, ksem0, ksem1, wsem0, wsem1)


def _stats_body(pc_ref, loads_ref, util_ref, exc_ref):
    pc = pc_ref[...]
    loads = jnp.sum(pc, axis=0, keepdims=True)
    loads_ref[...] = loads
    util_ref[...] = loads.astype(jnp.float32) * (1.0 / B)
    exc_ref[...] = (loads > CAPACITY).astype(jnp.int32)


_stats = pl.pallas_call(
    _stats_body,
    out_shape=(
        jax.ShapeDtypeStruct((1, E), jnp.int32),
        jax.ShapeDtypeStruct((1, E), jnp.float32),
        jax.ShapeDtypeStruct((1, E), jnp.int32),
    ),
)


def kernel(gates):
    rw, tki, tkw, pcnt = _router(gates.reshape(-1))
    loads, util, exc = _stats(pcnt.reshape(NW, E))
    return (
        rw.reshape(B, E),
        tki.reshape(B, K),
        util.reshape(E),
        loads.reshape(E),
        exc.reshape(E).astype(jnp.bool_),
        tkw.reshape(B, K),
    )


# speedup vs baseline: 1.0004x; 1.0004x over previous
import functools

import jax
import jax.numpy as jnp
from jax import lax
from jax.experimental import pallas as pl
from jax.experimental.pallas import tpu as pltpu
from jax.experimental.pallas import tpu_sc as plsc

B = 32768
E = 64
EP = E + 1
K = 8
NC, NS = 2, 16
NW = NC * NS
ROWS_PER_W = B // NW
CHUNK = 256
N_CHUNKS = ROWS_PER_W // CHUNK
GROUPS = CHUNK // 16
CAPACITY = int(B * 1.25 / E)
R_BLK = 1024

_NET19 = [(0, 1), (2, 3), (4, 5), (6, 7),
          (0, 2), (1, 3), (4, 6), (5, 7),
          (1, 2), (5, 6),
          (0, 4), (1, 5), (2, 6), (3, 7),
          (2, 4), (3, 5),
          (1, 2), (3, 4), (5, 6)]
_BITONIC12 = [(0, 4), (1, 5), (2, 6), (3, 7),
              (0, 2), (1, 3), (4, 6), (5, 7),
              (0, 1), (2, 3), (4, 5), (6, 7)]


def _router_body(gates_hbm, rw_hbm, tki_hbm, tkw_hbm, pcnt_hbm,
                 gbuf0, gbuf1, gbufp, rwbuf0, rwbuf1, tki0, tki1,
                 tkw0, tkw1, cntbuf,
                 isem0, isem1, rsem0, rsem1, ksem0, ksem1, wsem0, wsem1):
    wid = lax.axis_index("s") * NC + lax.axis_index("c")
    lane = lax.iota(jnp.int32, 16)
    zf = jnp.zeros((16,), jnp.float32)
    zi = jnp.zeros((16,), jnp.int32)
    gbufs, rwbufs = [gbuf0, gbuf1], [rwbuf0, rwbuf1]
    tkibufs, tkwbufs = [tki0, tki1], [tkw0, tkw1]
    isems, rsems = [isem0, isem1], [rsem0, rsem1]
    ksems, wsems = [ksem0, ksem1], [wsem0, wsem1]

    def gslice(c):
        return gates_hbm.at[pl.ds(wid * (ROWS_PER_W * E) + c * (CHUNK * E),
                                  CHUNK * E)]

    def rwslice(c):
        return rw_hbm.at[pl.ds(wid * (ROWS_PER_W * E) + c * (CHUNK * E),
                               CHUNK * E)]

    def kslice(ref, c):
        return ref.at[pl.ds(wid * (ROWS_PER_W * K) + c * (CHUNK * K),
                            CHUNK * K)]

    for q in range(E // 16):
        cntbuf[pl.ds(q * 16, 16)] = zi

    pltpu.make_async_copy(gslice(0), gbufs[0], isems[0]).start()

    def process_chunk(c, b, gbuf, rwbuf, tkibuf, tkwbuf):
        def repitch_body(r8, _):
            b64 = r8 * (8 * E)
            b65 = r8 * (8 * EP)
            for i in range(8):
                for q in range(E // 16):
                    gbufp[pl.ds(b65 + i * EP + q * 16, 16)] = (
                        gbuf[pl.ds(b64 + i * E + q * 16, 16)])
            return 0

        lax.fori_loop(0, CHUNK // 8, repitch_body, 0)

        def do_group(g):
            goff = g * (16 * E)
            goffp = g * (16 * EP)
            for r in range(16):
                for q in range(E // 16):
                    rwbuf[pl.ds(goff + r * E + q * 16, 16)] = zf

            rowaddr = goff + lane * E
            rowaddrp = goffp + lane * EP

            def keys_of(t):
                ks = []
                for j in range(8):
                    e = t * 8 + j
                    v = plsc.load_gather(gbufp, [rowaddrp + e])
                    bits = lax.bitcast_convert_type(v, jnp.int32)
                    kb = (bits & -64) | (63 - e)
                    ks.append(lax.bitcast_convert_type(kb, jnp.float32))
                return ks

            def sort8(v):
                for i, j in _NET19:
                    v[i], v[j] = jnp.maximum(v[i], v[j]), jnp.minimum(v[i], v[j])
                return v

            def merge_top8(a, b):
                c8 = [jnp.maximum(a[i], b[7 - i]) for i in range(8)]
                for i, j in _BITONIC12:
                    c8[i], c8[j] = (jnp.maximum(c8[i], c8[j]),
                                    jnp.minimum(c8[i], c8[j]))
                return c8

            m01 = merge_top8(sort8(keys_of(0)), sort8(keys_of(1)))
            m23 = merge_top8(sort8(keys_of(2)), sort8(keys_of(3)))
            m03 = merge_top8(m01, m23)
            m45 = merge_top8(sort8(keys_of(4)), sort8(keys_of(5)))
            m67 = merge_top8(sort8(keys_of(6)), sort8(keys_of(7)))
            m47 = merge_top8(m45, m67)
            f = merge_top8(m03, m47)

            mi = [63 - (lax.bitcast_convert_type(f[j], jnp.int32) & 63)
                  for j in range(K)]
            m = [plsc.load_gather(gbufp, [rowaddrp + mi[j]]) for j in range(K)]

            for j in range(K - 1):
                swp = m[j] < m[j + 1]
                hi = jnp.maximum(m[j], m[j + 1])
                lo = jnp.minimum(m[j], m[j + 1])
                hidx = jnp.where(swp, mi[j + 1], mi[j])
                lidx = jnp.where(swp, mi[j], mi[j + 1])
                m[j], m[j + 1] = hi, lo
                mi[j], mi[j + 1] = hidx, lidx

            p = [jnp.exp(m[j] - m[0]) for j in range(K)]
            s = p[0]
            for j in range(1, K):
                s = s + p[j]
            inv = 1.0 / s
            w = [p[j] * inv for j in range(K)]

            trow = g * (16 * K) + lane * K
            one = jnp.ones((16,), jnp.int32)
            for j in range(K):
                plsc.store_scatter(rwbuf, [rowaddr + mi[j]], w[j])
                plsc.store_scatter(tkwbuf, [trow + j], w[j])
                plsc.store_scatter(tkibuf, [trow + j], mi[j])
                plsc.addupdate_scatter(
                    cntbuf, [mi[j]], jnp.where(w[j] > 0.0, one, zi))

        def group_pair_body(g2, _):
            do_group(g2 * 2)
            do_group(g2 * 2 + 1)
            return 0

        lax.fori_loop(0, GROUPS // 2, group_pair_body, 0)

    def outer_body(c2, _):
        for b in range(2):
            c = c2 * 2 + b

            @pl.when(c + 1 < N_CHUNKS)
            def _():
                pltpu.make_async_copy(gslice(c + 1), gbufs[1 - b],
                                      isems[1 - b]).start()

            pltpu.make_async_copy(gslice(c), gbufs[b], isems[b]).wait()

            @pl.when(c >= 2)
            def _():
                pltpu.make_async_copy(rwbufs[b], rwslice(c - 2),
                                      rsems[b]).wait()
                pltpu.make_async_copy(tkibufs[b], kslice(tki_hbm, c - 2),
                                      ksems[b]).wait()
                pltpu.make_async_copy(tkwbufs[b], kslice(tkw_hbm, c - 2),
                                      wsems[b]).wait()

            process_chunk(c, b, gbufs[b], rwbufs[b], tkibufs[b], tkwbufs[b])

            pltpu.make_async_copy(rwbufs[b], rwslice(c), rsems[b]).start()
            pltpu.make_async_copy(tkibufs[b], kslice(tki_hbm, c),
                                  ksems[b]).start()
            pltpu.make_async_copy(tkwbufs[b], kslice(tkw_hbm, c),
                                  wsems[b]).start()
        return 0

    lax.fori_loop(0, N_CHUNKS // 2, outer_body, 0)

    for b in range(2):
        c = N_CHUNKS - 2 + b
        pltpu.make_async_copy(rwbufs[b], rwslice(c), rsems[b]).wait()
        pltpu.make_async_copy(tkibufs[b], kslice(tki_hbm, c), ksems[b]).wait()
        pltpu.make_async_copy(tkwbufs[b], kslice(tkw_hbm, c), wsems[b]).wait()
    pltpu.sync_copy(cntbuf, pcnt_hbm.at[pl.ds(wid * E, E)])


@functools.partial(
    pl.kernel,
    out_type=(
        jax.ShapeDtypeStruct((B * E,), jnp.float32),
        jax.ShapeDtypeStruct((B * K,), jnp.int32),
        jax.ShapeDtypeStruct((B * K,), jnp.float32),
        jax.ShapeDtypeStruct((NW * E,), jnp.int32),
    ),
    mesh=plsc.VectorSubcoreMesh(core_axis_name="c", subcore_axis_name="s"),
    compiler_params=pltpu.CompilerParams(needs_layout_passes=False),
    scratch_types=[
        pltpu.VMEM((CHUNK * E,), jnp.float32),
        pltpu.VMEM((CHUNK * E,), jnp.float32),
        pltpu.VMEM((CHUNK * EP,), jnp.float32),
        pltpu.VMEM((CHUNK * E,), jnp.float32),
        pltpu.VMEM((CHUNK * E,), jnp.float32),
        pltpu.VMEM((CHUNK * K,), jnp.int32),
        pltpu.VMEM((CHUNK * K,), jnp.int32),
        pltpu.VMEM((CHUNK * K,), jnp.float32),
        pltpu.VMEM((CHUNK * K,), jnp.float32),
        pltpu.VMEM((E,), jnp.int32),
        pltpu.SemaphoreType.DMA,
        pltpu.SemaphoreType.DMA,
        pltpu.SemaphoreType.DMA,
        pltpu.SemaphoreType.DMA,
        pltpu.SemaphoreType.DMA,
        pltpu.SemaphoreType.DMA,
        pltpu.SemaphoreType.DMA,
        pltpu.SemaphoreType.DMA,
    ],
)
def _router(gates_hbm, rw_hbm, tki_hbm, tkw_hbm, pcnt_hbm,
            gbuf0, gbuf1, gbufp, rwbuf0, rwbuf1, tki0, tki1,
            tkw0, tkw1, cntbuf,
            isem0, isem1, rsem0, rsem1, ksem0, ksem1, wsem0, wsem1):
    _router_body(gates_hbm, rw_hbm, tki_hbm, tkw_hbm, pcnt_hbm,
                 gbuf0, gbuf1, gbufp, rwbuf0, rwbuf1, tki0, tki1,
                 tkw0, tkw1, cntbuf,
                 isem0, isem1, rsem0, rsem1, ksem0, ksem1, wsem0, wsem1)


def _stats_body(pc_ref, loads_ref, util_ref, exc_ref):
    pc = pc_ref[...]
    loads = jnp.sum(pc, axis=0, keepdims=True)
    loads_ref[...] = loads
    util_ref[...] = loads.astype(jnp.float32) * (1.0 / B)
    exc_ref[...] = (loads > CAPACITY).astype(jnp.int32)


_stats = pl.pallas_call(
    _stats_body,
    out_shape=(
        jax.ShapeDtypeStruct((1, E), jnp.int32),
        jax.ShapeDtypeStruct((1, E), jnp.float32),
        jax.ShapeDtypeStruct((1, E), jnp.int32),
    ),
)


def kernel(gates):
    rw, tki, tkw, pcnt = _router(gates.reshape(-1))
    loads, util, exc = _stats(pcnt.reshape(NW, E))
    return (
        rw.reshape(B, E),
        tki.reshape(B, K),
        util.reshape(E),
        loads.reshape(E),
        exc.reshape(E).astype(jnp.bool_),
        tkw.reshape(B, K),
    )
